# baseline (speedup 1.0000x reference)
import jax
import jax.numpy as jnp
from jax.experimental import pallas as pl
from jax.experimental.pallas import tpu as pltpu


def _round_up(a: int, b: int) -> int:
    return (a + b - 1) // b * b


def _dinv_kernel(g_ref, dinv_ref, acc_ref):
    r = pl.program_id(1)

    @pl.when(r == 0)
    def _():
        acc_ref[...] = jnp.zeros_like(acc_ref)

    acc_ref[...] += jnp.sum(g_ref[...], axis=0, keepdims=True)

    @pl.when(r == pl.num_programs(1) - 1)
    def _():
        d = acc_ref[...]
        dinv = jnp.where(d > 0, jax.lax.rsqrt(d), 0.0)
        dinv_ref[...] = dinv.reshape(dinv_ref.shape)


def _hs_kernel(x_ref, w_ref, dinv_ref, hs_ref):
    h = jnp.dot(x_ref[...], w_ref[...], preferred_element_type=jnp.float32)
    hs_ref[...] = dinv_ref[...] * h


def _agg2_kernel(g0_ref, g1_ref, hs_ref, dinv_ref, b_ref, y_ref):
    tm = g0_ref.shape[1]
    hs = hs_ref[...]
    p0 = jax.lax.dot_general(
        g0_ref[...], hs,
        dimension_numbers=(((0,), (0,)), ((), ())),
        preferred_element_type=jnp.float32)
    p1 = jax.lax.dot_general(
        g1_ref[...], hs,
        dimension_numbers=(((0,), (0,)), ((), ())),
        preferred_element_type=jnp.float32)
    y_ref[:tm, :] = dinv_ref[:tm, :] * p0 + b_ref[...]
    y_ref[tm:, :] = dinv_ref[tm:, :] * p1 + b_ref[...]


def _agg1_kernel(g_ref, hs_ref, dinv_ref, b_ref, y_ref):
    prod = jax.lax.dot_general(
        g_ref[...], hs_ref[...],
        dimension_numbers=(((0,), (0,)), ((), ())),
        preferred_element_type=jnp.float32)
    y_ref[...] = dinv_ref[...] * prod + b_ref[...]


@jax.jit
def _gcn_forward(x, graph, weight, bias_row):
    N, M = x.shape
    F = weight.shape[1]

    x = x.astype(jnp.float32)
    graph = graph.astype(jnp.float32)
    weight = weight.astype(jnp.float32)

    LANE = 128
    Fp = _round_up(F, LANE)
    Np = _round_up(N, 512) if N >= 512 else _round_up(N, 8)

    CB = 4096 if Np % 4096 == 0 else Np
    RB = 256 if Np % 256 == 0 else Np

    if Np != N:
        xp = jnp.zeros((Np, M), jnp.float32).at[:N, :].set(x)
        gp = jnp.zeros((Np, Np), jnp.float32).at[:N, :N].set(graph)
    else:
        xp, gp = x, graph
    if Fp != F:
        wp = jnp.zeros((M, Fp), jnp.float32).at[:, :F].set(weight)
        bp = jnp.zeros((1, Fp), jnp.float32).at[:, :F].set(bias_row)
    else:
        wp, bp = weight, bias_row

    dinv_col = pl.pallas_call(
        _dinv_kernel,
        out_shape=jax.ShapeDtypeStruct((Np, 1), jnp.float32),
        grid=(Np // CB, Np // RB),
        in_specs=[pl.BlockSpec((RB, CB), lambda c, r: (r, c))],
        out_specs=pl.BlockSpec((CB, 1), lambda c, r: (c, 0)),
        scratch_shapes=[pltpu.VMEM((1, CB), jnp.float32)],
        compiler_params=pltpu.CompilerParams(
            dimension_semantics=("parallel", "arbitrary")),
    )(gp)

    TB = 2048 if Np % 2048 == 0 else Np
    hs = pl.pallas_call(
        _hs_kernel,
        out_shape=jax.ShapeDtypeStruct((Np, Fp), jnp.float32),
        grid=(Np // TB,),
        in_specs=[
            pl.BlockSpec((TB, M), lambda i: (i, 0)),
            pl.BlockSpec((M, Fp), lambda i: (0, 0)),
            pl.BlockSpec((TB, 1), lambda i: (i, 0)),
        ],
        out_specs=pl.BlockSpec((TB, Fp), lambda i: (i, 0)),
        compiler_params=pltpu.CompilerParams(
            dimension_semantics=("parallel",)),
    )(xp, wp, dinv_col)

    if Np % 512 == 0 and Np * 256 * 4 * 4 + Np * Fp * 4 < 48 * 1024 * 1024:
        TM = 256
        y_padded = pl.pallas_call(
            _agg2_kernel,
            out_shape=jax.ShapeDtypeStruct((Np, Fp), jnp.float32),
            grid=(Np // (2 * TM),),
            in_specs=[
                pl.BlockSpec((Np, TM), lambda i: (0, 2 * i)),
                pl.BlockSpec((Np, TM), lambda i: (0, 2 * i + 1)),
                pl.BlockSpec((Np, Fp), lambda i: (0, 0)),
                pl.BlockSpec((2 * TM, 1), lambda i: (i, 0)),
                pl.BlockSpec((1, Fp), lambda i: (0, 0)),
            ],
            out_specs=pl.BlockSpec((2 * TM, Fp), lambda i: (i, 0)),
            compiler_params=pltpu.CompilerParams(
                dimension_semantics=("parallel",)),
        )(gp, gp, hs, dinv_col, bp)
    else:
        TM = 256 if Np % 256 == 0 else Np
        y_padded = pl.pallas_call(
            _agg1_kernel,
            out_shape=jax.ShapeDtypeStruct((Np, Fp), jnp.float32),
            grid=(Np // TM,),
            in_specs=[
                pl.BlockSpec((Np, TM), lambda i: (0, i)),
                pl.BlockSpec((Np, Fp), lambda i: (0, 0)),
                pl.BlockSpec((TM, 1), lambda i: (i, 0)),
                pl.BlockSpec((1, Fp), lambda i: (0, 0)),
            ],
            out_specs=pl.BlockSpec((TM, Fp), lambda i: (i, 0)),
            compiler_params=pltpu.CompilerParams(
                dimension_semantics=("parallel",)),
        )(gp, hs, dinv_col, bp)

    return y_padded[:N, :F]


def kernel(x, graph, weight, bias):
    F = weight.shape[1]
    if bias is None:
        bias_row = jnp.zeros((1, F), jnp.float32)
    else:
        bias_row = bias.astype(jnp.float32).reshape(1, F)
    return _gcn_forward(x, graph, weight, bias_row)


# speedup vs baseline: 1.0002x; 1.0002x over previous
import jax
import jax.numpy as jnp
from jax.experimental import pallas as pl
from jax.experimental.pallas import tpu as pltpu


def _round_up(a: int, b: int) -> int:
    return (a + b - 1) // b * b


def _dinv_kernel(g_ref, dinv_ref, acc_ref):
    r = pl.program_id(1)

    @pl.when(r == 0)
    def _():
        acc_ref[...] = jnp.zeros_like(acc_ref)

    acc_ref[...] += jnp.sum(g_ref[...], axis=0, keepdims=True)

    @pl.when(r == pl.num_programs(1) - 1)
    def _():
        d = acc_ref[...]
        dinv = jnp.where(d > 0, jax.lax.rsqrt(d), 0.0)
        dinv_ref[...] = dinv.reshape(dinv_ref.shape)


def _hs_kernel(x_ref, w_ref, dinv_ref, hs_ref):
    h = jnp.dot(x_ref[...], w_ref[...], preferred_element_type=jnp.float32)
    hs_ref[...] = dinv_ref[...] * h


def _agg2_kernel(g0_ref, g1_ref, hs_ref, dinv_ref, b_ref, y_ref):
    tm = g0_ref.shape[1]
    n = g0_ref.shape[0]
    ck = min(2048, n)
    dn = (((0,), (0,)), ((), ()))
    p0 = None
    p1 = None
    for c in range(n // ck):
        lo = c * ck
        hs_c = hs_ref[lo:lo + ck, :]
        d0 = jax.lax.dot_general(g0_ref[lo:lo + ck, :], hs_c, dn,
                                 preferred_element_type=jnp.float32)
        d1 = jax.lax.dot_general(g1_ref[lo:lo + ck, :], hs_c, dn,
                                 preferred_element_type=jnp.float32)
        p0 = d0 if p0 is None else p0 + d0
        p1 = d1 if p1 is None else p1 + d1
    y_ref[:tm, :] = dinv_ref[:tm, :] * p0 + b_ref[...]
    y_ref[tm:, :] = dinv_ref[tm:, :] * p1 + b_ref[...]


def _agg1_kernel(g_ref, hs_ref, dinv_ref, b_ref, y_ref):
    prod = jax.lax.dot_general(
        g_ref[...], hs_ref[...],
        dimension_numbers=(((0,), (0,)), ((), ())),
        preferred_element_type=jnp.float32)
    y_ref[...] = dinv_ref[...] * prod + b_ref[...]


@jax.jit
def _gcn_forward(x, graph, weight, bias_row):
    N, M = x.shape
    F = weight.shape[1]

    x = x.astype(jnp.float32)
    graph = graph.astype(jnp.float32)
    weight = weight.astype(jnp.float32)

    LANE = 128
    Fp = _round_up(F, LANE)
    Np = _round_up(N, 512) if N >= 512 else _round_up(N, 8)

    CB = 4096 if Np % 4096 == 0 else Np
    RB = 256 if Np % 256 == 0 else Np

    if Np != N:
        xp = jnp.zeros((Np, M), jnp.float32).at[:N, :].set(x)
        gp = jnp.zeros((Np, Np), jnp.float32).at[:N, :N].set(graph)
    else:
        xp, gp = x, graph
    if Fp != F:
        wp = jnp.zeros((M, Fp), jnp.float32).at[:, :F].set(weight)
        bp = jnp.zeros((1, Fp), jnp.float32).at[:, :F].set(bias_row)
    else:
        wp, bp = weight, bias_row

    dinv_col = pl.pallas_call(
        _dinv_kernel,
        out_shape=jax.ShapeDtypeStruct((Np, 1), jnp.float32),
        grid=(Np // CB, Np // RB),
        in_specs=[pl.BlockSpec((RB, CB), lambda c, r: (r, c))],
        out_specs=pl.BlockSpec((CB, 1), lambda c, r: (c, 0)),
        scratch_shapes=[pltpu.VMEM((1, CB), jnp.float32)],
        compiler_params=pltpu.CompilerParams(
            dimension_semantics=("parallel", "arbitrary")),
    )(gp)

    TB = 2048 if Np % 2048 == 0 else Np
    hs = pl.pallas_call(
        _hs_kernel,
        out_shape=jax.ShapeDtypeStruct((Np, Fp), jnp.float32),
        grid=(Np // TB,),
        in_specs=[
            pl.BlockSpec((TB, M), lambda i: (i, 0)),
            pl.BlockSpec((M, Fp), lambda i: (0, 0)),
            pl.BlockSpec((TB, 1), lambda i: (i, 0)),
        ],
        out_specs=pl.BlockSpec((TB, Fp), lambda i: (i, 0)),
        compiler_params=pltpu.CompilerParams(
            dimension_semantics=("parallel",)),
    )(xp, wp, dinv_col)

    if Np % 512 == 0 and Np * 256 * 4 * 4 + Np * Fp * 4 < 48 * 1024 * 1024:
        TM = 256
        y_padded = pl.pallas_call(
            _agg2_kernel,
            out_shape=jax.ShapeDtypeStruct((Np, Fp), jnp.float32),
            grid=(Np // (2 * TM),),
            in_specs=[
                pl.BlockSpec((Np, TM), lambda i: (0, 2 * i)),
                pl.BlockSpec((Np, TM), lambda i: (0, 2 * i + 1)),
                pl.BlockSpec((Np, Fp), lambda i: (0, 0)),
                pl.BlockSpec((2 * TM, 1), lambda i: (i, 0)),
                pl.BlockSpec((1, Fp), lambda i: (0, 0)),
            ],
            out_specs=pl.BlockSpec((2 * TM, Fp), lambda i: (i, 0)),
            compiler_params=pltpu.CompilerParams(
                dimension_semantics=("parallel",)),
        )(gp, gp, hs, dinv_col, bp)
    else:
        TM = 256 if Np % 256 == 0 else Np
        y_padded = pl.pallas_call(
            _agg1_kernel,
            out_shape=jax.ShapeDtypeStruct((Np, Fp), jnp.float32),
            grid=(Np // TM,),
            in_specs=[
                pl.BlockSpec((Np, TM), lambda i: (0, i)),
                pl.BlockSpec((Np, Fp), lambda i: (0, 0)),
                pl.BlockSpec((TM, 1), lambda i: (i, 0)),
                pl.BlockSpec((1, Fp), lambda i: (0, 0)),
            ],
            out_specs=pl.BlockSpec((TM, Fp), lambda i: (i, 0)),
            compiler_params=pltpu.CompilerParams(
                dimension_semantics=("parallel",)),
        )(gp, hs, dinv_col, bp)

    return y_padded[:N, :F]


def kernel(x, graph, weight, bias):
    F = weight.shape[1]
    if bias is None:
        bias_row = jnp.zeros((1, F), jnp.float32)
    else:
        bias_row = bias.astype(jnp.float32).reshape(1, F)
    return _gcn_forward(x, graph, weight, bias_row)
